# baseline (speedup 1.0000x reference)
import jax
import jax.numpy as jnp
from jax import lax
from jax.experimental import pallas as pl
from jax.experimental.pallas import tpu as pltpu
from jax.experimental.pallas import tpu_sc as plsc

_N = 100000
_D = 128
_H = 10000
_GS = 32

_NC, _NS = 2, 16
_CH = 8
_CR = _CH * _GS
_NV = _D // 16

_NCH0 = 56
_NCH1 = 24
_HPW0 = _NCH0 * _CH
_HPW1 = _NCH1 * _CH
_HPS = _HPW0 + _HPW1
_HPAD = _NS * _HPS
_HIDX = _HPAD + _HPW0 - _HPW1 + 2 * _CH


def _mm_body(ne_ref, wt_ref, b_ref, x_ref):
    x_ref[...] = jnp.maximum(
        jnp.dot(ne_ref[...].astype(jnp.bfloat16),
                wt_ref[...].astype(jnp.bfloat16),
                preferred_element_type=jnp.float32)
        + b_ref[...], 0.0)


def _transform(ne, wt, b):
    bn = 2000
    return pl.pallas_call(
        _mm_body,
        grid=(_N // bn,),
        in_specs=[
            pl.BlockSpec((bn, _D), lambda i: (i, 0)),
            pl.BlockSpec((_D, _D), lambda i: (0, 0)),
            pl.BlockSpec((1, _D), lambda i: (0, 0)),
        ],
        out_specs=pl.BlockSpec((bn, _D), lambda i: (i, 0)),
        out_shape=jax.ShapeDtypeStruct((_N, _D), jnp.float32),
    )(ne, wt, b.reshape(1, _D))


def _sc_body(x_hbm, idx_hbm, out_hbm, idx_v, buf,
             zero_v, didx0, didx1, spacc, tmp_v, out8_v,
             s00, s01, s10, s11):
    c = lax.axis_index("c")
    s = lax.axis_index("s")
    off = s * _HPS + c * _HPW0
    nch = _NCH0 - c * (_NCH0 - _NCH1)
    pltpu.sync_copy(idx_hbm.at[pl.ds(off * _GS, (_NCH0 + 1) * _CR)], idx_v)

    for i in range(2 * _CH):
        for d in range(_NV):
            zero_v[i, pl.ds(d * 16, 16)] = jnp.zeros((16,), jnp.float32)
    lane16 = lax.iota(jnp.int32, 16)
    for i in range(8):
        didx0[pl.ds(i * 16, 16)] = lane16 + s * (2 * _CH)
        didx1[pl.ds(i * 16, 16)] = lane16 + s * (2 * _CH)

    half = _CR // 2
    sems = ((s00, s01), (s10, s11))

    def gather(k, b):
        return (pltpu.async_copy(
                    x_hbm.at[idx_v.at[pl.ds(k * _CR, half)]],
                    buf.at[b].at[pl.ds(0, half)], sems[b][0]),
                pltpu.async_copy(
                    x_hbm.at[idx_v.at[pl.ds(k * _CR + half, half)]],
                    buf.at[b].at[pl.ds(half, half)], sems[b][1]))

    gather(0, 0)

    def pair(p, carry):
        for bb in range(2):
            k = 2 * p + bb
            pltpu.make_async_copy(
                x_hbm.at[idx_v.at[pl.ds(k * _CR, half)]],
                buf.at[bb].at[pl.ds(0, half)], sems[bb][0]).wait()
            pltpu.make_async_copy(
                x_hbm.at[idx_v.at[pl.ds(k * _CR + half, half)]],
                buf.at[bb].at[pl.ds(half, half)], sems[bb][1]).wait()
            pltpu.sync_copy(zero_v, spacc.at[pl.ds(s * 2 * _CH, 2 * _CH)])
            pltpu.sync_copy(buf.at[bb].at[pl.ds(0, half)],
                            spacc.at[didx0], add=True)
            pltpu.sync_copy(buf.at[bb].at[pl.ds(half, half)],
                            spacc.at[didx1], add=True)
            gather(k + 1, 1 - bb)
            pltpu.sync_copy(spacc.at[pl.ds(s * 2 * _CH, 2 * _CH)], tmp_v)
            for j in range(_CH):
                for d in range(_NV):
                    o = d * 16
                    out8_v[j, pl.ds(o, 16)] = (
                        tmp_v[j, pl.ds(o, 16)] + tmp_v[j + _CH, pl.ds(o, 16)])
            pltpu.sync_copy(out8_v, out_hbm.at[pl.ds(off + k * _CH, _CH)])
        return carry

    lax.fori_loop(0, nch // 2, pair, 0)
    pltpu.make_async_copy(
        x_hbm.at[idx_v.at[pl.ds(nch * _CR, half)]],
        buf.at[0].at[pl.ds(0, half)], sems[0][0]).wait()
    pltpu.make_async_copy(
        x_hbm.at[idx_v.at[pl.ds(nch * _CR + half, half)]],
        buf.at[0].at[pl.ds(half, half)], sems[0][1]).wait()


def _aggregate(x, idx):
    mesh = plsc.VectorSubcoreMesh(core_axis_name="c", subcore_axis_name="s")
    k = pl.kernel(
        _sc_body,
        out_type=jax.ShapeDtypeStruct((_HPAD, _D), jnp.float32),
        mesh=mesh,
        scratch_types=[
            pltpu.VMEM(((_NCH0 + 1) * _CR,), jnp.int32),
            pltpu.VMEM((2, _CR, _D), jnp.float32),
            pltpu.VMEM((2 * _CH, _D), jnp.float32),
            pltpu.VMEM((_CR // 2,), jnp.int32),
            pltpu.VMEM((_CR // 2,), jnp.int32),
            pltpu.VMEM_SHARED((_NS * 2 * _CH, _D), jnp.float32),
            pltpu.VMEM((2 * _CH, _D), jnp.float32),
            pltpu.VMEM((_CH, _D), jnp.float32),
            pltpu.SemaphoreType.DMA,
            pltpu.SemaphoreType.DMA,
            pltpu.SemaphoreType.DMA,
            pltpu.SemaphoreType.DMA,
        ],
    )
    return k(x, idx)


def kernel(node_embeddings, hyperedges, hyperedge_subsets, W, b):
    del hyperedges
    x = _transform(node_embeddings, W.T * (1.0 / _GS), b * (1.0 / _GS))
    idx = hyperedge_subsets.astype(jnp.int32).reshape(_H, _GS)
    idx = jnp.pad(idx, ((0, _HIDX - _H), (0, 0)))
    idx = idx.reshape(_HIDX // _CH, _CH, _GS).transpose(0, 2, 1)
    idx = idx.reshape(_HIDX * _GS)
    return _aggregate(x, idx)[:_H]


# speedup vs baseline: 1.0636x; 1.0636x over previous
import jax
import jax.numpy as jnp
from jax import lax
from jax.experimental import pallas as pl
from jax.experimental.pallas import tpu as pltpu
from jax.experimental.pallas import tpu_sc as plsc

_N = 100000
_D = 128
_H = 10000
_GS = 32

_NC, _NS = 2, 16
_CH = 8
_CR = _CH * _GS
_NV = _D // 16

_NCH0 = 56
_NCH1 = 24
_HPW0 = _NCH0 * _CH
_HPW1 = _NCH1 * _CH
_HPS = _HPW0 + _HPW1
_HPAD = _NS * _HPS
_HIDX = _HPAD + _HPW0 - _HPW1


def _mm_body(ne_ref, wt_ref, b_ref, x_ref):
    x_ref[...] = jnp.maximum(
        jnp.dot(ne_ref[...].astype(jnp.bfloat16),
                wt_ref[...].astype(jnp.bfloat16),
                preferred_element_type=jnp.float32)
        + b_ref[...], 0.0)


def _transform(ne, wt, b):
    bn = 2000
    return pl.pallas_call(
        _mm_body,
        grid=(_N // bn,),
        in_specs=[
            pl.BlockSpec((bn, _D), lambda i: (i, 0)),
            pl.BlockSpec((_D, _D), lambda i: (0, 0)),
            pl.BlockSpec((1, _D), lambda i: (0, 0)),
        ],
        out_specs=pl.BlockSpec((bn, _D), lambda i: (i, 0)),
        out_shape=jax.ShapeDtypeStruct((_N, _D), jnp.float32),
    )(ne, wt, b.reshape(1, _D))


def _sc_body(x_hbm, idx_hbm, out_hbm, idx_v, buf, out_v, sem, sem2):
    c = lax.axis_index("c")
    s = lax.axis_index("s")
    off = s * _HPS + c * _HPW0
    nch = _NCH0 - c * (_NCH0 - _NCH1)
    pltpu.sync_copy(idx_hbm.at[pl.ds(off * _GS, _NCH0 * _CR)], idx_v)

    def reduce_chunk(k):
        for h in range(_CH):
            base = h * _GS
            row = k * _CH + h
            for d in range(_NV):
                o = d * 16
                vals = [buf[base + r, pl.ds(o, 16)] for r in range(_GS)]
                while len(vals) > 1:
                    vals = [vals[i] + vals[i + 1]
                            for i in range(0, len(vals), 2)]
                out_v[row, pl.ds(o, 16)] = vals[0] * (1.0 / _GS)

    half = _CR // 2

    def chunk(k, carry):
        cp1 = pltpu.async_copy(
            x_hbm.at[idx_v.at[pl.ds(k * _CR, half)]],
            buf.at[pl.ds(0, half)], sem)
        cp2 = pltpu.async_copy(
            x_hbm.at[idx_v.at[pl.ds(k * _CR + half, half)]],
            buf.at[pl.ds(half, half)], sem2)
        cp1.wait()
        cp2.wait()
        reduce_chunk(k)
        return carry

    lax.fori_loop(0, nch, chunk, 0)

    @pl.when(c == 0)
    def _():
        pltpu.sync_copy(out_v, out_hbm.at[pl.ds(off, _HPW0)])

    @pl.when(c == 1)
    def _():
        pltpu.sync_copy(out_v.at[pl.ds(0, _HPW1)],
                        out_hbm.at[pl.ds(off, _HPW1)])


def _aggregate(x, idx):
    mesh = plsc.VectorSubcoreMesh(core_axis_name="c", subcore_axis_name="s")
    k = pl.kernel(
        _sc_body,
        out_type=jax.ShapeDtypeStruct((_HPAD, _D), jnp.float32),
        mesh=mesh,
        scratch_types=[
            pltpu.VMEM((_NCH0 * _CR,), jnp.int32),
            pltpu.VMEM((_CR, _D), jnp.float32),
            pltpu.VMEM((_HPW0, _D), jnp.float32),
            pltpu.SemaphoreType.DMA,
            pltpu.SemaphoreType.DMA,
        ],
    )
    return k(x, idx)


def kernel(node_embeddings, hyperedges, hyperedge_subsets, W, b):
    del hyperedges
    x = _transform(node_embeddings, W.T, b)
    idx = hyperedge_subsets.astype(jnp.int32).reshape(_H, _GS)
    idx = jnp.pad(idx, ((0, _HIDX - _H), (0, 0)))
    idx = idx.reshape(_HIDX * _GS)
    return _aggregate(x, idx)[:_H]
